# baseline (speedup 1.0000x reference)
import jax
import jax.numpy as jnp
from jax import lax
from jax.experimental import pallas as pl
from jax.experimental.pallas import tpu as pltpu
from jax.experimental.pallas import tpu_sc as plsc

NC = 2
NS = 16
NW = NC * NS
LANES = 16

B = 16384
N_ITEMS = 5
NAME_LEN = 8
ROWS = B * N_ITEMS
BATCH_PW = B // NW
ITEMS_PW = ROWS // NW
IW = 80
CB = 16
CHUNK = CB * N_ITEMS
NCH = BATCH_PW // CB
TOKR = NAME_LEN * CHUNK // IW
D_ID = 16
D_GICS = 16
D_NAME = 32
D_OUT = 64
PAD_SL = 8
PAD_LN = 128


def _body(id_idx, gics_idx, name_idx, id_table, gics_table, name_table,
          out, idx_id_v, idx_gics_v, idx_name_v,
          id_r0, id_r1, gc_r0, gc_r1, tok0, tok1, st0, st1,
          sem_g0, sem_g1, sem_o0, sem_o1):
    wid = lax.axis_index("s") * NC + lax.axis_index("c")
    bufs = ((id_r0, gc_r0, tok0, st0, sem_g0, sem_o0),
            (id_r1, gc_r1, tok1, st1, sem_g1, sem_o1))

    pltpu.sync_copy(id_idx.at[pl.ds(wid * NCH, NCH)], idx_id_v)
    pltpu.sync_copy(gics_idx.at[pl.ds(wid * NCH, NCH)], idx_gics_v)
    pltpu.sync_copy(name_idx.at[pl.ds(wid * NCH * TOKR, NCH * TOKR)],
                    idx_name_v)

    def gather_descs(ci, b):
        idr, gcr, tok, _, semg, _ = bufs[b]
        ds = [pltpu.make_async_copy(id_table.at[idx_id_v.at[ci]], idr, semg),
              pltpu.make_async_copy(gics_table.at[idx_gics_v.at[ci]], gcr,
                                    semg)]
        for t in range(TOKR):
            ds.append(pltpu.make_async_copy(
                name_table.at[idx_name_v.at[ci * TOKR + t]],
                tok.at[pl.ds(t * IW, IW)], semg))
        return ds

    def out_desc(ci, b):
        _, _, _, st, _, semo = bufs[b]
        return pltpu.make_async_copy(
            st, out.at[pl.ds(wid * BATCH_PW + ci * CB, CB)], semo)

    def pool_repack(b):
        idr, gcr, tok, st, _, _ = bufs[b]

        def pool_body(i, _):
            for j in range(N_ITEMS):
                r = i * N_ITEMS + j
                st[i, j, pl.ds(0, LANES)] = idr[r, pl.ds(0, LANES)]
                st[i, j, pl.ds(D_ID, LANES)] = gcr[r, pl.ds(0, LANES)]
                s0 = tok[r * NAME_LEN, pl.ds(0, LANES)]
                s1 = tok[r * NAME_LEN, pl.ds(LANES, LANES)]
                for t in range(1, NAME_LEN):
                    s0 = s0 + tok[r * NAME_LEN + t, pl.ds(0, LANES)]
                    s1 = s1 + tok[r * NAME_LEN + t, pl.ds(LANES, LANES)]
                st[i, j, pl.ds(D_ID + D_GICS, LANES)] = s0 * (1.0 / NAME_LEN)
                st[i, j, pl.ds(D_ID + D_GICS + LANES, LANES)] = (
                    s1 * (1.0 / NAME_LEN))
            return 0

        lax.fori_loop(0, CB, pool_body, 0)

    for d in gather_descs(0, 0):
        d.start()

    def super_body(s, _):
        for b in range(2):
            ci = 2 * s + b
            nb = 1 - b

            @pl.when(ci + 1 < NCH)
            def _issue_next():
                @pl.when(ci >= 1)
                def _drain_prev_out():
                    out_desc(ci - 1, nb).wait()

                for d in gather_descs(ci + 1, nb):
                    d.start()

            for d in gather_descs(ci, b):
                d.wait()
            pool_repack(b)
            out_desc(ci, b).start()
        return 0

    lax.fori_loop(0, NCH // 2, super_body, 0)
    out_desc(NCH - 2, 0).wait()
    out_desc(NCH - 1, 1).wait()


def _fmt_body(x_ref, o_ref):
    o_ref[...] = x_ref[:, :N_ITEMS, :D_OUT]


@jax.jit
def kernel(item_id, item_gics, item_name_tokens, id_table, gics_table,
           name_table):
    id_idx = jnp.asarray(item_id, jnp.int32).reshape(ROWS // IW, IW)
    gics_idx = jnp.asarray(item_gics, jnp.int32).reshape(ROWS // IW, IW)
    name_idx = jnp.asarray(item_name_tokens, jnp.int32).reshape(
        ROWS * NAME_LEN // IW, IW)

    kfn = pl.kernel(
        _body,
        out_type=jax.ShapeDtypeStruct((B, PAD_SL, PAD_LN), jnp.float32),
        mesh=plsc.VectorSubcoreMesh(core_axis_name="c", subcore_axis_name="s"),
        compiler_params=pltpu.CompilerParams(use_tc_tiling_on_sc=False),
        scratch_types=[
            pltpu.VMEM((NCH, IW), jnp.int32),
            pltpu.VMEM((NCH, IW), jnp.int32),
            pltpu.VMEM((NCH * TOKR, IW), jnp.int32),
            pltpu.VMEM((CHUNK, D_ID), jnp.float32),
            pltpu.VMEM((CHUNK, D_ID), jnp.float32),
            pltpu.VMEM((CHUNK, D_GICS), jnp.float32),
            pltpu.VMEM((CHUNK, D_GICS), jnp.float32),
            pltpu.VMEM((CHUNK * NAME_LEN, D_NAME), jnp.float32),
            pltpu.VMEM((CHUNK * NAME_LEN, D_NAME), jnp.float32),
            pltpu.VMEM((CB, PAD_SL, PAD_LN), jnp.float32),
            pltpu.VMEM((CB, PAD_SL, PAD_LN), jnp.float32),
            pltpu.SemaphoreType.DMA,
            pltpu.SemaphoreType.DMA,
            pltpu.SemaphoreType.DMA,
            pltpu.SemaphoreType.DMA,
        ],
    )
    out_padded = kfn(id_idx, gics_idx, name_idx, id_table, gics_table,
                     name_table)

    return pl.pallas_call(
        _fmt_body,
        grid=(64,),
        in_specs=[pl.BlockSpec((B // 64, PAD_SL, PAD_LN),
                               lambda i: (i, 0, 0))],
        out_specs=pl.BlockSpec((B // 64, N_ITEMS, D_OUT),
                               lambda i: (i, 0, 0)),
        out_shape=jax.ShapeDtypeStruct((B, N_ITEMS, D_OUT), jnp.float32),
    )(out_padded)
